# baseline (speedup 1.0000x reference)
import jax
import jax.numpy as jnp
from jax import lax
from jax.experimental import pallas as pl
from jax.experimental.pallas import tpu as pltpu
from jax.experimental.pallas import tpu_sc as plsc

NUM_PRIM = 8192
DIM = 64
BATCH = 64
HW = 1024
N = BATCH * HW
NC, NS = 2, 16
NW = NC * NS
CHUNK = 128
PER_W = N // NW
NCH = PER_W // CHUNK
CSLICE = NUM_PRIM // NS
LANES = 16
OUT_ROWS = N * DIM // 128


def _sc_body(idx_g, table, emb_out, hist_out,
             idx_v, rows_v, ones_v, zeros_v, hist_sh, sem0, sem1):
    c = lax.axis_index("c")
    s = lax.axis_index("s")
    wid = s * NC + c

    pltpu.sync_copy(idx_g.at[wid], idx_v)

    one = jnp.ones((LANES,), jnp.int32)
    zero = jnp.zeros((LANES,), jnp.int32)
    for i in range(CHUNK // LANES):
        ones_v[pl.ds(i * LANES, LANES)] = one
    for i in range(CSLICE // LANES):
        zeros_v[pl.ds(i * LANES, LANES)] = zero
    pltpu.sync_copy(zeros_v, hist_sh.at[pl.ds(s * CSLICE, CSLICE)])

    plsc.subcore_barrier()

    def hstep(j, carry):
        pltpu.sync_copy(ones_v, hist_sh.at[idx_v.at[j]], add=True)
        return carry
    lax.fori_loop(0, NCH, hstep, 0)

    def gpair(p, carry):
        k0 = p * 2
        d0 = pltpu.async_copy(table.at[idx_v.at[k0]], rows_v.at[0], sem0)
        d1 = pltpu.async_copy(table.at[idx_v.at[k0 + 1]], rows_v.at[1], sem1)
        d0.wait()
        pltpu.sync_copy(rows_v.at[0], emb_out.at[wid * NCH + k0])
        d1.wait()
        pltpu.sync_copy(rows_v.at[1], emb_out.at[wid * NCH + k0 + 1])
        return carry

    lax.fori_loop(0, NCH // 2, gpair, 0)

    plsc.subcore_barrier()

    sl = pl.ds(s * CSLICE, CSLICE)
    pltpu.sync_copy(hist_sh.at[sl], hist_out.at[c].at[sl])


_sc_kernel = pl.kernel(
    _sc_body,
    out_type=(
        jax.ShapeDtypeStruct((NW * NCH, CHUNK, DIM), jnp.float32),
        jax.ShapeDtypeStruct((NC, NUM_PRIM), jnp.int32),
    ),
    mesh=plsc.VectorSubcoreMesh(
        core_axis_name="c", subcore_axis_name="s",
        num_cores=NC, num_subcores=NS,
    ),
    compiler_params=pltpu.CompilerParams(use_tc_tiling_on_sc=False),
    scratch_types=[
        pltpu.VMEM((NCH, CHUNK), jnp.int32),
        pltpu.VMEM((2, CHUNK, DIM), jnp.float32),
        pltpu.VMEM((CHUNK,), jnp.int32),
        pltpu.VMEM((CSLICE,), jnp.int32),
        pltpu.VMEM_SHARED((NUM_PRIM,), jnp.int32),
        pltpu.SemaphoreType.DMA,
        pltpu.SemaphoreType.DMA,
    ],
)

B_BLK = 4
TC_GRID = BATCH // B_BLK
ROWS_PER_B = HW * DIM // 128


def _tc_body(emb_ref, hist_ref, act_ref, out_ref, cnt_ref):
    for bb in range(B_BLK):
        x = emb_ref[pl.ds(bb * ROWS_PER_B, ROWS_PER_B), :]
        y = jnp.stack([x[:, 0:DIM].T, x[:, DIM:2 * DIM].T], axis=-1)
        out_ref[bb] = y.reshape(DIM, HW)

    @pl.when(pl.program_id(0) == 0)
    def _():
        cnt_ref[...] = act_ref[...] + hist_ref[0] + hist_ref[1]


_tc_kernel = pl.pallas_call(
    _tc_body,
    grid=(TC_GRID,),
    in_specs=[
        pl.BlockSpec((B_BLK * ROWS_PER_B, 128), lambda i: (i, 0)),
        pl.BlockSpec((NC, 64, 128), lambda i: (0, 0, 0)),
        pl.BlockSpec((64, 128), lambda i: (0, 0)),
    ],
    out_specs=[
        pl.BlockSpec((B_BLK, DIM, HW), lambda i: (i, 0, 0)),
        pl.BlockSpec((64, 128), lambda i: (0, 0)),
    ],
    out_shape=(
        jax.ShapeDtypeStruct((BATCH, DIM, HW), jnp.float32),
        jax.ShapeDtypeStruct((64, 128), jnp.int32),
    ),
)


@jax.jit
def kernel(indices, primitives, activation_count):
    idx_g = indices.reshape(NW, NCH, CHUNK)
    emb_sc, hist = _sc_kernel(idx_g, primitives)
    out_t, cnt2d = _tc_kernel(
        emb_sc.reshape(OUT_ROWS, 128),
        hist.reshape(NC, 64, 128),
        activation_count.reshape(64, 128),
    )
    return out_t.transpose(0, 2, 1), cnt2d.reshape(NUM_PRIM)


# speedup vs baseline: 13.1894x; 13.1894x over previous
import jax
import jax.numpy as jnp
from jax import lax
from jax.experimental import pallas as pl
from jax.experimental.pallas import tpu as pltpu
from jax.experimental.pallas import tpu_sc as plsc

NUM_PRIM = 8192
DIM = 64
BATCH = 64
HW = 1024
N = BATCH * HW
NC, NS = 2, 16
NW = NC * NS
CHUNK = 128
PER_W = N // NW
NCH = PER_W // CHUNK
CSLICE = NUM_PRIM // NS
LANES = 16
OUT_ROWS = N * DIM // 128


def _sc_body(idx_g, table, emb_out, hist_out,
             idx_v, rows_v, ones_v, zeros_v, hist_sh, sem0, sem1):
    c = lax.axis_index("c")
    s = lax.axis_index("s")
    wid = s * NC + c

    pltpu.sync_copy(idx_g.at[wid], idx_v)

    one = jnp.ones((LANES,), jnp.int32)
    zero = jnp.zeros((LANES,), jnp.int32)
    for i in range(CHUNK // LANES):
        ones_v[pl.ds(i * LANES, LANES)] = one
    for i in range(CSLICE // LANES):
        zeros_v[pl.ds(i * LANES, LANES)] = zero
    pltpu.sync_copy(zeros_v, hist_sh.at[pl.ds(s * CSLICE, CSLICE)])

    plsc.subcore_barrier()

    def hstep(j, carry):
        pltpu.sync_copy(ones_v, hist_sh.at[idx_v.at[j]], add=True)
        return carry
    lax.fori_loop(0, NCH, hstep, 0)

    def gpair(p, carry):
        k0 = p * 2
        d0 = pltpu.async_copy(table.at[idx_v.at[k0]], rows_v.at[0], sem0)
        d1 = pltpu.async_copy(table.at[idx_v.at[k0 + 1]], rows_v.at[1], sem1)
        d0.wait()
        pltpu.sync_copy(rows_v.at[0], emb_out.at[wid * NCH + k0])
        d1.wait()
        pltpu.sync_copy(rows_v.at[1], emb_out.at[wid * NCH + k0 + 1])
        return carry

    lax.fori_loop(0, NCH // 2, gpair, 0)

    plsc.subcore_barrier()

    sl = pl.ds(s * CSLICE, CSLICE)
    pltpu.sync_copy(hist_sh.at[sl], hist_out.at[c].at[sl])


_sc_kernel = pl.kernel(
    _sc_body,
    out_type=(
        jax.ShapeDtypeStruct((NW * NCH, CHUNK, DIM), jnp.float32),
        jax.ShapeDtypeStruct((NC, NUM_PRIM), jnp.int32),
    ),
    mesh=plsc.VectorSubcoreMesh(
        core_axis_name="c", subcore_axis_name="s",
        num_cores=NC, num_subcores=NS,
    ),
    compiler_params=pltpu.CompilerParams(use_tc_tiling_on_sc=False),
    scratch_types=[
        pltpu.VMEM((NCH, CHUNK), jnp.int32),
        pltpu.VMEM((2, CHUNK, DIM), jnp.float32),
        pltpu.VMEM((CHUNK,), jnp.int32),
        pltpu.VMEM((CSLICE,), jnp.int32),
        pltpu.VMEM_SHARED((NUM_PRIM,), jnp.int32),
        pltpu.SemaphoreType.DMA,
        pltpu.SemaphoreType.DMA,
    ],
)

B_BLK = 4
TC_GRID = BATCH // B_BLK
ROWS_PER_B = HW * DIM // 128


def _tc_body(emb_ref, hist_ref, act_ref, out_ref, cnt_ref):
    for bb in range(B_BLK):
        x = emb_ref[pl.ds(bb * ROWS_PER_B, ROWS_PER_B), :]
        out_ref[bb] = jnp.concatenate(
            [x[:, 0:DIM].T, x[:, DIM:2 * DIM].T], axis=1)

    @pl.when(pl.program_id(0) == 0)
    def _():
        cnt_ref[...] = act_ref[...] + hist_ref[0] + hist_ref[1]


_tc_kernel = pl.pallas_call(
    _tc_body,
    grid=(TC_GRID,),
    in_specs=[
        pl.BlockSpec((B_BLK * ROWS_PER_B, 128), lambda i: (i, 0)),
        pl.BlockSpec((NC, 64, 128), lambda i: (0, 0, 0)),
        pl.BlockSpec((64, 128), lambda i: (0, 0)),
    ],
    out_specs=[
        pl.BlockSpec((B_BLK, DIM, HW), lambda i: (i, 0, 0)),
        pl.BlockSpec((64, 128), lambda i: (0, 0)),
    ],
    out_shape=(
        jax.ShapeDtypeStruct((BATCH, DIM, HW), jnp.float32),
        jax.ShapeDtypeStruct((64, 128), jnp.int32),
    ),
)


@jax.jit
def kernel(indices, primitives, activation_count):
    idx_perm = indices.reshape(BATCH, 2, HW // 2).swapaxes(1, 2)
    idx_g = idx_perm.reshape(NW, NCH, CHUNK)
    emb_sc, hist = _sc_kernel(idx_g, primitives)
    out_t, cnt2d = _tc_kernel(
        emb_sc.reshape(OUT_ROWS, 128),
        hist.reshape(NC, 64, 128),
        activation_count.reshape(64, 128),
    )
    return out_t.transpose(0, 2, 1), cnt2d.reshape(NUM_PRIM)


# speedup vs baseline: 17.3462x; 1.3152x over previous
import jax
import jax.numpy as jnp
from jax import lax
from jax.experimental import pallas as pl
from jax.experimental.pallas import tpu as pltpu
from jax.experimental.pallas import tpu_sc as plsc

NUM_PRIM = 8192
DIM = 64
BATCH = 64
HW = 1024
N = BATCH * HW
NC, NS = 2, 16
NW = NC * NS
CHUNK = 128
PER_W = N // NW
NCH = PER_W // CHUNK
CSLICE = NUM_PRIM // NS
LANES = 16
OUT_ROWS = N * DIM // 128


def _sc_body(idx_g, table, emb_out, hist_out,
             idx_v, rows_v, ones_v, zeros_v, hist_sh, sem0, sem1):
    c = lax.axis_index("c")
    s = lax.axis_index("s")
    wid = s * NC + c

    pltpu.sync_copy(idx_g.at[wid], idx_v)

    one = jnp.ones((LANES,), jnp.int32)
    zero = jnp.zeros((LANES,), jnp.int32)
    for i in range(CHUNK // LANES):
        ones_v[pl.ds(i * LANES, LANES)] = one
    for i in range(CSLICE // LANES):
        zeros_v[pl.ds(i * LANES, LANES)] = zero
    pltpu.sync_copy(zeros_v, hist_sh.at[pl.ds(s * CSLICE, CSLICE)])

    plsc.subcore_barrier()

    def hstep(j, carry):
        pltpu.sync_copy(ones_v, hist_sh.at[idx_v.at[j]], add=True)
        return carry
    lax.fori_loop(0, NCH, hstep, 0)

    def gpair(p, carry):
        k0 = p * 2
        d0 = pltpu.async_copy(table.at[idx_v.at[k0]], rows_v.at[0], sem0)
        d1 = pltpu.async_copy(table.at[idx_v.at[k0 + 1]], rows_v.at[1], sem1)
        d0.wait()
        pltpu.sync_copy(rows_v.at[0], emb_out.at[wid * NCH + k0])
        d1.wait()
        pltpu.sync_copy(rows_v.at[1], emb_out.at[wid * NCH + k0 + 1])
        return carry

    lax.fori_loop(0, NCH // 2, gpair, 0)

    plsc.subcore_barrier()

    sl = pl.ds(s * CSLICE, CSLICE)
    pltpu.sync_copy(hist_sh.at[sl], hist_out.at[c].at[sl])


_sc_kernel = pl.kernel(
    _sc_body,
    out_type=(
        jax.ShapeDtypeStruct((NW * NCH, CHUNK, DIM), jnp.float32),
        jax.ShapeDtypeStruct((NC, NUM_PRIM), jnp.int32),
    ),
    mesh=plsc.VectorSubcoreMesh(
        core_axis_name="c", subcore_axis_name="s",
        num_cores=NC, num_subcores=NS,
    ),
    compiler_params=pltpu.CompilerParams(use_tc_tiling_on_sc=False),
    scratch_types=[
        pltpu.VMEM((NCH, CHUNK), jnp.int32),
        pltpu.VMEM((2, CHUNK, DIM), jnp.float32),
        pltpu.VMEM((CHUNK,), jnp.int32),
        pltpu.VMEM((CSLICE,), jnp.int32),
        pltpu.VMEM_SHARED((NUM_PRIM,), jnp.int32),
        pltpu.SemaphoreType.DMA,
        pltpu.SemaphoreType.DMA,
    ],
)

B_BLK = 8
TC_GRID = BATCH // B_BLK
ROWS_PER_B = HW * DIM // 128


def _tc_body(emb_ref, hist_ref, act_ref, out_ref, cnt_ref):
    for bb in range(B_BLK):
        x = emb_ref[pl.ds(bb * ROWS_PER_B, ROWS_PER_B), :]
        out_ref[bb] = jnp.concatenate(
            [x[:, 0:DIM].T, x[:, DIM:2 * DIM].T], axis=1)

    @pl.when(pl.program_id(0) == 0)
    def _():
        cnt_ref[...] = act_ref[...] + hist_ref[0] + hist_ref[1]


_tc_kernel = pl.pallas_call(
    _tc_body,
    grid=(TC_GRID,),
    in_specs=[
        pl.BlockSpec((B_BLK * ROWS_PER_B, 128), lambda i: (i, 0)),
        pl.BlockSpec((NC, 64, 128), lambda i: (0, 0, 0)),
        pl.BlockSpec((64, 128), lambda i: (0, 0)),
    ],
    out_specs=[
        pl.BlockSpec((B_BLK, DIM, HW), lambda i: (i, 0, 0)),
        pl.BlockSpec((64, 128), lambda i: (0, 0)),
    ],
    out_shape=(
        jax.ShapeDtypeStruct((BATCH, DIM, HW), jnp.float32),
        jax.ShapeDtypeStruct((64, 128), jnp.int32),
    ),
)


@jax.jit
def kernel(indices, primitives, activation_count):
    hh = jnp.arange(HW, dtype=jnp.int32)
    perm = (hh % 2) * (HW // 2) + hh // 2
    idx_perm = jnp.take(indices, perm, axis=1)
    idx_g = idx_perm.reshape(NW, NCH, CHUNK)
    emb_sc, hist = _sc_kernel(idx_g, primitives)
    out_t, cnt2d = _tc_kernel(
        emb_sc.reshape(OUT_ROWS, 128),
        hist.reshape(NC, 64, 128),
        activation_count.reshape(64, 128),
    )
    return out_t.transpose(0, 2, 1), cnt2d.reshape(NUM_PRIM)


# speedup vs baseline: 19.0269x; 1.0969x over previous
import jax
import jax.numpy as jnp
from jax import lax
from jax.experimental import pallas as pl
from jax.experimental.pallas import tpu as pltpu
from jax.experimental.pallas import tpu_sc as plsc

NUM_PRIM = 8192
DIM = 64
BATCH = 64
HW = 1024
N = BATCH * HW
NC, NS = 2, 16
NW = NC * NS
CHUNK = 128
PER_W = N // NW
NCH = PER_W // CHUNK
CSLICE = NUM_PRIM // NS
LANES = 16
OUT_ROWS = N * DIM // 128


NBUF = 12


def _sc_body(idx_g, table, emb_out, hist_out,
             idx_v, rows_v, ones_v, zeros_v, hist_sh, gsem, ssem, hsem):
    c = lax.axis_index("c")
    s = lax.axis_index("s")
    wid = s * NC + c

    pltpu.sync_copy(idx_g.at[wid], idx_v)

    one = jnp.ones((LANES,), jnp.int32)
    zero = jnp.zeros((LANES,), jnp.int32)
    for i in range(CHUNK // LANES):
        ones_v[pl.ds(i * LANES, LANES)] = one
    for i in range(CSLICE // LANES):
        zeros_v[pl.ds(i * LANES, LANES)] = zero
    pltpu.sync_copy(zeros_v, hist_sh.at[pl.ds(s * CSLICE, CSLICE)])

    plsc.subcore_barrier()

    def hstep(j, carry):
        pltpu.async_copy(ones_v, hist_sh.at[idx_v.at[j]], hsem, add=True)
        return carry
    lax.fori_loop(0, NCH, hstep, 0)

    def prime(k, carry):
        pltpu.async_copy(table.at[idx_v.at[k]], rows_v.at[k], gsem)
        return carry
    lax.fori_loop(0, NBUF, prime, 0)

    def gstep(k, carry):
        slot = k % NBUF
        pltpu.make_async_copy(table.at[pl.ds(0, CHUNK)],
                              rows_v.at[slot], gsem).wait()
        pltpu.async_copy(rows_v.at[slot], emb_out.at[wid * NCH + k], ssem)

        @pl.when(k + NBUF < NCH)
        def _():
            pltpu.make_async_copy(table.at[pl.ds(0, CHUNK)],
                                  rows_v.at[slot], ssem).wait()
            pltpu.async_copy(table.at[idx_v.at[k + NBUF]],
                             rows_v.at[slot], gsem)
        return carry
    lax.fori_loop(0, NCH, gstep, 0)

    def sdrain(k, carry):
        pltpu.make_async_copy(table.at[pl.ds(0, CHUNK)],
                              rows_v.at[0], ssem).wait()
        return carry
    lax.fori_loop(0, NBUF, sdrain, 0)

    pltpu.make_async_copy(idx_g.at[0], idx_v, hsem).wait()

    plsc.subcore_barrier()

    sl = pl.ds(s * CSLICE, CSLICE)
    pltpu.sync_copy(hist_sh.at[sl], hist_out.at[c].at[sl])


_sc_kernel = pl.kernel(
    _sc_body,
    out_type=(
        jax.ShapeDtypeStruct((NW * NCH, CHUNK, DIM), jnp.float32),
        jax.ShapeDtypeStruct((NC, NUM_PRIM), jnp.int32),
    ),
    mesh=plsc.VectorSubcoreMesh(
        core_axis_name="c", subcore_axis_name="s",
        num_cores=NC, num_subcores=NS,
    ),
    compiler_params=pltpu.CompilerParams(use_tc_tiling_on_sc=False),
    scratch_types=[
        pltpu.VMEM((NCH, CHUNK), jnp.int32),
        pltpu.VMEM((NBUF, CHUNK, DIM), jnp.float32),
        pltpu.VMEM((CHUNK,), jnp.int32),
        pltpu.VMEM((CSLICE,), jnp.int32),
        pltpu.VMEM_SHARED((NUM_PRIM,), jnp.int32),
        pltpu.SemaphoreType.DMA,
        pltpu.SemaphoreType.DMA,
        pltpu.SemaphoreType.DMA,
    ],
)

B_BLK = 16
TC_GRID = BATCH // B_BLK
ROWS_PER_B = HW * DIM // 128


def _tc_body(emb_ref, hist_ref, act_ref, out_ref, cnt_ref):
    for bb in range(B_BLK):
        x = emb_ref[pl.ds(bb * ROWS_PER_B, ROWS_PER_B), :]
        out_ref[bb] = jnp.concatenate(
            [x[:, 0:DIM].T, x[:, DIM:2 * DIM].T], axis=1)

    @pl.when(pl.program_id(0) == 0)
    def _():
        cnt_ref[...] = act_ref[...] + hist_ref[0] + hist_ref[1]


_tc_kernel = pl.pallas_call(
    _tc_body,
    grid=(TC_GRID,),
    in_specs=[
        pl.BlockSpec((B_BLK * ROWS_PER_B, 128), lambda i: (i, 0)),
        pl.BlockSpec((NC, 64, 128), lambda i: (0, 0, 0)),
        pl.BlockSpec((64, 128), lambda i: (0, 0)),
    ],
    out_specs=[
        pl.BlockSpec((B_BLK, DIM, HW), lambda i: (i, 0, 0)),
        pl.BlockSpec((64, 128), lambda i: (0, 0)),
    ],
    out_shape=(
        jax.ShapeDtypeStruct((BATCH, DIM, HW), jnp.float32),
        jax.ShapeDtypeStruct((64, 128), jnp.int32),
    ),
)


@jax.jit
def kernel(indices, primitives, activation_count):
    hh = jnp.arange(HW, dtype=jnp.int32)
    perm = (hh % 2) * (HW // 2) + hh // 2
    idx_perm = jnp.take(indices, perm, axis=1)
    idx_g = idx_perm.reshape(NW, NCH, CHUNK)
    emb_sc, hist = _sc_kernel(idx_g, primitives)
    out_t, cnt2d = _tc_kernel(
        emb_sc.reshape(OUT_ROWS, 128),
        hist.reshape(NC, 64, 128),
        activation_count.reshape(64, 128),
    )
    return out_t.transpose(0, 2, 1), cnt2d.reshape(NUM_PRIM)
